# baseline (speedup 1.0000x reference)
import jax
import jax.numpy as jnp
from jax.experimental import pallas as pl

BT = 1024


def _add_block(x_ref, e_ref, o_ref):
    o_ref[...] = x_ref[...] + e_ref[...]


def kernel(x, emb):
    B, T, E = x.shape
    return pl.pallas_call(
        _add_block,
        grid=(T // BT, B),
        in_specs=[
            pl.BlockSpec((1, BT, E), lambda i, j: (j, i, 0)),
            pl.BlockSpec((BT, E), lambda i, j: (i, 0)),
        ],
        out_specs=pl.BlockSpec((1, BT, E), lambda i, j: (j, i, 0)),
        out_shape=jax.ShapeDtypeStruct((B, T, E), x.dtype),
    )(x, emb)
